# baseline (speedup 1.0000x reference)
import dataclasses
import functools
from math import gcd as _gcd

import jax
import jax.numpy as jnp
from jax import lax
from jax.experimental import pallas as pl
from jax.experimental.pallas import tpu as pltpu
from jax.experimental.pallas import tpu_sc as plsc

_CHUNK = 80
_NTILES = 32
_SUBCORES = 16
_LANES = 16


def _sc_edge_aggregate(table, src, dst, npad, n, with_deg, chunk, nbuf, nib):
    v, width = table.shape
    e = src.shape[0]
    ncr = -(-e // chunk)
    niter = -(-ncr // _NTILES)
    if ncr * chunk > e:
        pad = ncr * chunk - e
        src = jnp.concatenate([src, jnp.zeros((pad,), jnp.int32)])
        dst = jnp.concatenate([dst, jnp.full((pad,), n + 8, jnp.int32)])
    rows_per_tile = npad // _SUBCORES

    mesh = plsc.VectorSubcoreMesh(core_axis_name="c", subcore_axis_name="s")

    out_type = [jax.ShapeDtypeStruct((2 * npad, width), jnp.float32)]
    scratch = [pltpu.VMEM((chunk,), jnp.int32)] * (2 * nib)
    scratch += [pltpu.VMEM((chunk, width), jnp.float32)] * nbuf
    scratch += [
        pltpu.VMEM_SHARED((npad, width), jnp.float32),
    ]
    scratch += [pltpu.SemaphoreType.DMA] * (nbuf + nib + nbuf)
    if with_deg:
        out_type.append(jax.ShapeDtypeStruct((_NTILES, npad), jnp.float32))
        scratch.append(pltpu.VMEM((npad,), jnp.float32))

    cp = pltpu.CompilerParams()
    if "needs_layout_passes" in pltpu.CompilerParams.__dataclass_fields__:
        cp = dataclasses.replace(cp, needs_layout_passes=False)

    @functools.partial(pl.kernel, out_type=out_type, mesh=mesh,
                       scratch_types=scratch, compiler_params=cp)
    def agg_kernel(table_hbm, src_hbm, dst_hbm, *refs):
        if with_deg:
            out_hbm, deg_hbm = refs[0], refs[1]
            rest = refs[2:-1]
            ldeg = refs[-1]
        else:
            out_hbm = refs[0]
            rest = refs[1:]
        sidx = rest[0:nib]
        didx = rest[nib:2 * nib]
        rows = rest[2 * nib:2 * nib + nbuf]
        shared = rest[2 * nib + nbuf]
        sems = rest[2 * nib + nbuf + 1:]
        semg = sems[0:nbuf]
        semi = sems[nbuf:nbuf + nib]
        semsc = sems[nbuf + nib:]
        cid = lax.axis_index("c")
        sid = lax.axis_index("s")
        wid = sid * 2 + cid
        cbase = wid * niter * chunk

        def idx_copies(j, q):
            base = cbase + j * chunk
            return (pltpu.make_async_copy(src_hbm.at[pl.ds(base, chunk)],
                                          sidx[q], semi[q]),
                    pltpu.make_async_copy(dst_hbm.at[pl.ds(base, chunk)],
                                          didx[q], semi[q]))

        def gather(q, b):
            return pltpu.make_async_copy(table_hbm.at[sidx[q]],
                                         rows[b], semg[b])

        def scatter(q, b):
            return pltpu.make_async_copy(rows[b], shared.at[didx[q]],
                                         semsc[b])

        myreal = jnp.clip(ncr - wid * niter, 0, niter)

        for q in range(nib - 1):
            @pl.when(q < myreal)
            def _():
                for c in idx_copies(q, q):
                    c.start()

        zv = jnp.zeros((_LANES,), jnp.float32)

        @pl.loop(0, chunk)
        def _(i):
            for k in range(width // _LANES):
                rows[0][i, pl.ds(k * _LANES, _LANES)] = zv

        base_r = sid * rows_per_tile
        for f in range(rows_per_tile // chunk):
            pltpu.sync_copy(rows[0],
                            shared.at[pl.ds(base_r + f * chunk, chunk)])
        rem = rows_per_tile % chunk
        if rem:
            pltpu.sync_copy(
                rows[0].at[pl.ds(0, rem)],
                shared.at[pl.ds(base_r + rows_per_tile - rem, rem)])
        if with_deg:
            @pl.loop(0, npad // _LANES)
            def _(i):
                ldeg[pl.ds(i * _LANES, _LANES)] = zv

        plsc.subcore_barrier()

        for js in range(nbuf - 1):
            @pl.when(js < myreal)
            def _():
                for c in idx_copies(js, js):
                    c.wait()
                gather(js, js).start()

        ones = jnp.ones((_LANES,), jnp.float32)
        unroll = nib * nbuf // _gcd(nib, nbuf)

        @pl.loop(0, -(-niter // unroll))
        def _(jj):
            for u in range(unroll):
                j = jj * unroll + u
                q = u % nib
                r = u % nbuf

                @pl.when(j < myreal)
                def _():
                    gather(q, r).wait()
                    scatter(q, r).start(add=True)
                    if with_deg:
                        for k in range(chunk // _LANES):
                            idxv = didx[q][pl.ds(k * _LANES, _LANES)]
                            plsc.addupdate_scatter(ldeg, [idxv], ones)

                    @pl.when(j + nbuf - 1 < myreal)
                    def _():
                        for c in idx_copies(j + nbuf - 1,
                                            (q + nbuf - 1) % nib):
                            c.wait()

                        @pl.when(j >= 1)
                        def _():
                            scatter((q + nib - 1) % nib,
                                    (q + nbuf - 1) % nbuf).wait()

                        gather((q + nbuf - 1) % nib,
                               (q + nbuf - 1) % nbuf).start()

                        @pl.when(j + nib - 1 < myreal)
                        def _():
                            for c in idx_copies(j + nib - 1,
                                                (q + nib - 1) % nib):
                                c.start()

        for s in range(nbuf):
            @pl.when(myreal > s)
            def _():
                scatter(0, s).wait()

        plsc.subcore_barrier()
        pltpu.sync_copy(shared.at[pl.ds(base_r, rows_per_tile)],
                        out_hbm.at[pl.ds(cid * npad + base_r, rows_per_tile)])
        if with_deg:
            pltpu.sync_copy(ldeg, deg_hbm.at[wid])

    return agg_kernel(table, src, dst)


def _tc_layer1_body(pa_ref, dp_ref, w_ref, b_ref, h_ref, dinv_ref):
    npad = pa_ref.shape[0] // 2
    s = pa_ref[:npad, :] + pa_ref[npad:, :]
    ones = jnp.ones((_NTILES, 1), jnp.float32)
    deg = lax.dot_general(dp_ref[...], ones, (((0,), (0,)), ((), ())),
                          preferred_element_type=jnp.float32)
    dinv = 1.0 / jnp.maximum(deg, 1.0)
    dinv_ref[...] = dinv
    z = jnp.dot(s * dinv, w_ref[...], preferred_element_type=jnp.float32)
    h_ref[...] = jnp.maximum(z + b_ref[...], 0.0)


def _tc_layer2_body(pb_ref, dinv_ref, batch_ref, w_ref, b_ref, wo_ref, bo_ref,
                    out_ref, *, num_graphs):
    npad = pb_ref.shape[0] // 2
    s = pb_ref[:npad, :] + pb_ref[npad:, :]
    h = jnp.maximum(
        jnp.dot(s * dinv_ref[...], w_ref[...],
                preferred_element_type=jnp.float32) + b_ref[...], 0.0)
    b = batch_ref[...]
    gids = lax.broadcasted_iota(jnp.int32, (1, num_graphs), 1)
    pt = (b == gids).astype(jnp.float32)
    counts = jnp.maximum(jnp.sum(pt, axis=0), 1.0)
    hg = lax.dot_general(pt, h, (((0,), (0,)), ((), ())),
                         preferred_element_type=jnp.float32)
    hg = hg / counts[:, None]
    out_ref[...] = jnp.dot(hg, wo_ref[...],
                           preferred_element_type=jnp.float32) + bo_ref[...]


def kernel(x, edge_index, batch, W1, b1, W2, b2, Wout, bout):
    n, d = x.shape
    num_graphs = 64
    npad = ((n + _NTILES * 8 - 1) // (_NTILES * 8)) * (_NTILES * 8)

    src = edge_index[0]
    dst = edge_index[1]

    pa, dp = _sc_edge_aggregate(x, src, dst, npad, n, with_deg=True,
                                chunk=64, nbuf=4, nib=8)
    h1, dinv = pl.pallas_call(
        _tc_layer1_body,
        out_shape=[jax.ShapeDtypeStruct((npad, 128), jnp.float32),
                   jax.ShapeDtypeStruct((npad, 1), jnp.float32)],
    )(pa, dp, W1, b1)

    (pb,) = _sc_edge_aggregate(h1, src, dst, npad, n, with_deg=False,
                               chunk=80, nbuf=4, nib=8)

    batch_p = jnp.concatenate(
        [batch, jnp.full((npad - n,), num_graphs, jnp.int32)]).reshape(npad, 1)
    out = pl.pallas_call(
        functools.partial(_tc_layer2_body, num_graphs=num_graphs),
        out_shape=jax.ShapeDtypeStruct((num_graphs, 128), jnp.float32),
    )(pb, dinv, batch_p, W2, b2, Wout, bout)
    return out


# speedup vs baseline: 1.0049x; 1.0049x over previous
import dataclasses
import functools
from math import gcd as _gcd

import jax
import jax.numpy as jnp
from jax import lax
from jax.experimental import pallas as pl
from jax.experimental.pallas import tpu as pltpu
from jax.experimental.pallas import tpu_sc as plsc

_CHUNK = 80
_NTILES = 32
_SUBCORES = 16
_LANES = 16


def _sc_edge_aggregate(table, src, dst, npad, n, with_deg, chunk, nbuf, nib):
    v, width = table.shape
    e = src.shape[0]
    ncr = -(-e // chunk)
    niter = -(-ncr // _NTILES)
    if ncr * chunk > e:
        pad = ncr * chunk - e
        src = jnp.concatenate([src, jnp.zeros((pad,), jnp.int32)])
        dst = jnp.concatenate([dst, jnp.full((pad,), n + 8, jnp.int32)])
    rows_per_tile = npad // _SUBCORES

    mesh = plsc.VectorSubcoreMesh(core_axis_name="c", subcore_axis_name="s")

    out_type = [jax.ShapeDtypeStruct((2 * npad, width), jnp.float32)]
    scratch = [pltpu.VMEM((chunk,), jnp.int32)] * (2 * nib)
    scratch += [pltpu.VMEM((chunk, width), jnp.float32)] * nbuf
    scratch += [
        pltpu.VMEM_SHARED((npad, width), jnp.float32),
    ]
    scratch += [pltpu.SemaphoreType.DMA] * (nbuf + nib + nbuf)
    if with_deg:
        out_type.append(jax.ShapeDtypeStruct((_NTILES, npad), jnp.float32))
        scratch.append(pltpu.VMEM((npad,), jnp.float32))

    cp = pltpu.CompilerParams()
    if "needs_layout_passes" in pltpu.CompilerParams.__dataclass_fields__:
        cp = dataclasses.replace(cp, needs_layout_passes=False)

    @functools.partial(pl.kernel, out_type=out_type, mesh=mesh,
                       scratch_types=scratch, compiler_params=cp)
    def agg_kernel(table_hbm, src_hbm, dst_hbm, *refs):
        if with_deg:
            out_hbm, deg_hbm = refs[0], refs[1]
            rest = refs[2:-1]
            ldeg = refs[-1]
        else:
            out_hbm = refs[0]
            rest = refs[1:]
        sidx = rest[0:nib]
        didx = rest[nib:2 * nib]
        rows = rest[2 * nib:2 * nib + nbuf]
        shared = rest[2 * nib + nbuf]
        sems = rest[2 * nib + nbuf + 1:]
        semg = sems[0:nbuf]
        semi = sems[nbuf:nbuf + nib]
        semsc = sems[nbuf + nib:]
        cid = lax.axis_index("c")
        sid = lax.axis_index("s")
        wid = sid * 2 + cid
        cbase = wid * niter * chunk

        def idx_copies(j, q):
            base = cbase + j * chunk
            return (pltpu.make_async_copy(src_hbm.at[pl.ds(base, chunk)],
                                          sidx[q], semi[q]),
                    pltpu.make_async_copy(dst_hbm.at[pl.ds(base, chunk)],
                                          didx[q], semi[q]))

        def gather(q, b):
            return pltpu.make_async_copy(table_hbm.at[sidx[q]],
                                         rows[b], semg[b])

        def scatter(q, b):
            return pltpu.make_async_copy(rows[b], shared.at[didx[q]],
                                         semsc[b])

        myreal = jnp.clip(ncr - wid * niter, 0, niter)

        for q in range(nib - 1):
            @pl.when(q < myreal)
            def _():
                for c in idx_copies(q, q):
                    c.start()

        zv = jnp.zeros((_LANES,), jnp.float32)

        @pl.loop(0, chunk)
        def _(i):
            for k in range(width // _LANES):
                rows[0][i, pl.ds(k * _LANES, _LANES)] = zv

        base_r = sid * rows_per_tile
        for f in range(rows_per_tile // chunk):
            pltpu.sync_copy(rows[0],
                            shared.at[pl.ds(base_r + f * chunk, chunk)])
        rem = rows_per_tile % chunk
        if rem:
            pltpu.sync_copy(
                rows[0].at[pl.ds(0, rem)],
                shared.at[pl.ds(base_r + rows_per_tile - rem, rem)])
        if with_deg:
            @pl.loop(0, npad // _LANES)
            def _(i):
                ldeg[pl.ds(i * _LANES, _LANES)] = zv

        plsc.subcore_barrier()

        for js in range(nbuf - 1):
            @pl.when(js < myreal)
            def _():
                for c in idx_copies(js, js):
                    c.wait()
                gather(js, js).start()

        ones = jnp.ones((_LANES,), jnp.float32)
        unroll = nib * nbuf // _gcd(nib, nbuf)

        @pl.loop(0, -(-niter // unroll))
        def _(jj):
            for u in range(unroll):
                j = jj * unroll + u
                q = u % nib
                r = u % nbuf

                @pl.when(j < myreal)
                def _():
                    gather(q, r).wait()
                    scatter(q, r).start(add=True)
                    if with_deg:
                        for k in range(chunk // _LANES):
                            idxv = didx[q][pl.ds(k * _LANES, _LANES)]
                            plsc.addupdate_scatter(ldeg, [idxv], ones)

                    @pl.when(j + nbuf - 1 < myreal)
                    def _():
                        for c in idx_copies(j + nbuf - 1,
                                            (q + nbuf - 1) % nib):
                            c.wait()

                        @pl.when(j >= 1)
                        def _():
                            scatter((q + nib - 1) % nib,
                                    (q + nbuf - 1) % nbuf).wait()

                        gather((q + nbuf - 1) % nib,
                               (q + nbuf - 1) % nbuf).start()

                        @pl.when(j + nib - 1 < myreal)
                        def _():
                            for c in idx_copies(j + nib - 1,
                                                (q + nib - 1) % nib):
                                c.start()

        for s in range(nbuf):
            @pl.when(myreal > s)
            def _():
                scatter(0, s).wait()

        plsc.subcore_barrier()
        pltpu.sync_copy(shared.at[pl.ds(base_r, rows_per_tile)],
                        out_hbm.at[pl.ds(cid * npad + base_r, rows_per_tile)])
        if with_deg:
            pltpu.sync_copy(ldeg, deg_hbm.at[wid])

    return agg_kernel(table, src, dst)


def _tc_layer1_body(pa_ref, dp_ref, w_ref, b_ref, h_ref, dinv_ref):
    npad = pa_ref.shape[0] // 2
    s = pa_ref[:npad, :] + pa_ref[npad:, :]
    ones = jnp.ones((_NTILES, 1), jnp.float32)
    deg = lax.dot_general(dp_ref[...], ones, (((0,), (0,)), ((), ())),
                          preferred_element_type=jnp.float32)
    dinv = 1.0 / jnp.maximum(deg, 1.0)
    dinv_ref[...] = dinv
    z = jnp.dot(s * dinv, w_ref[...], preferred_element_type=jnp.float32)
    h_ref[...] = jnp.maximum(z + b_ref[...], 0.0)


def _tc_layer2_body(pb_ref, dinv_ref, batch_ref, w_ref, b_ref, wo_ref, bo_ref,
                    out_ref, *, num_graphs):
    npad = pb_ref.shape[0] // 2
    s = pb_ref[:npad, :] + pb_ref[npad:, :]
    h = jnp.maximum(
        jnp.dot(s * dinv_ref[...], w_ref[...],
                preferred_element_type=jnp.float32) + b_ref[...], 0.0)
    b = batch_ref[...]
    gids = lax.broadcasted_iota(jnp.int32, (1, num_graphs), 1)
    pt = (b == gids).astype(jnp.float32)
    counts = jnp.maximum(jnp.sum(pt, axis=0), 1.0)
    hg = lax.dot_general(pt, h, (((0,), (0,)), ((), ())),
                         preferred_element_type=jnp.float32)
    hg = hg / counts[:, None]
    out_ref[...] = jnp.dot(hg, wo_ref[...],
                           preferred_element_type=jnp.float32) + bo_ref[...]


def kernel(x, edge_index, batch, W1, b1, W2, b2, Wout, bout):
    n, d = x.shape
    num_graphs = 64
    npad = ((n + _NTILES * 8 - 1) // (_NTILES * 8)) * (_NTILES * 8)

    src = edge_index[0]
    dst = edge_index[1]

    pa, dp = _sc_edge_aggregate(x, src, dst, npad, n, with_deg=True,
                                chunk=80, nbuf=3, nib=6)
    h1, dinv = pl.pallas_call(
        _tc_layer1_body,
        out_shape=[jax.ShapeDtypeStruct((npad, 128), jnp.float32),
                   jax.ShapeDtypeStruct((npad, 1), jnp.float32)],
    )(pa, dp, W1, b1)

    (pb,) = _sc_edge_aggregate(h1, src, dst, npad, n, with_deg=False,
                               chunk=80, nbuf=4, nib=8)

    batch_p = jnp.concatenate(
        [batch, jnp.full((npad - n,), num_graphs, jnp.int32)]).reshape(npad, 1)
    out = pl.pallas_call(
        functools.partial(_tc_layer2_body, num_graphs=num_graphs),
        out_shape=jax.ShapeDtypeStruct((num_graphs, 128), jnp.float32),
    )(pb, dinv, batch_p, W2, b2, Wout, bout)
    return out


# speedup vs baseline: 1.0258x; 1.0207x over previous
import dataclasses
import functools
from math import gcd as _gcd

import jax
import jax.numpy as jnp
from jax import lax
from jax.experimental import pallas as pl
from jax.experimental.pallas import tpu as pltpu
from jax.experimental.pallas import tpu_sc as plsc

_CHUNK = 80
_NTILES = 32
_SUBCORES = 16
_LANES = 16


def _sc_edge_aggregate(table, src, dst, npad, n, with_deg, chunk, nbuf, nib):
    v, width = table.shape
    e = src.shape[0]
    ncr = -(-e // chunk)
    niter = -(-ncr // _NTILES)
    if ncr * chunk > e:
        pad = ncr * chunk - e
        src = jnp.concatenate([src, jnp.zeros((pad,), jnp.int32)])
        dst = jnp.concatenate([dst, jnp.full((pad,), n + 8, jnp.int32)])
    rows_per_tile = npad // _SUBCORES

    mesh = plsc.VectorSubcoreMesh(core_axis_name="c", subcore_axis_name="s")

    out_type = [jax.ShapeDtypeStruct((2 * npad, width), jnp.float32)]
    scratch = [pltpu.VMEM((chunk,), jnp.int32)] * (2 * nib)
    scratch += [pltpu.VMEM((chunk, width), jnp.float32)] * nbuf
    scratch += [
        pltpu.VMEM_SHARED((npad, width), jnp.float32),
    ]
    scratch += [pltpu.SemaphoreType.DMA] * (nbuf + nib + nbuf)
    if with_deg:
        out_type.append(jax.ShapeDtypeStruct((_NTILES, npad), jnp.float32))
        scratch.append(pltpu.VMEM((npad,), jnp.float32))

    cp = pltpu.CompilerParams()
    if "needs_layout_passes" in pltpu.CompilerParams.__dataclass_fields__:
        cp = dataclasses.replace(cp, needs_layout_passes=False)

    @functools.partial(pl.kernel, out_type=out_type, mesh=mesh,
                       scratch_types=scratch, compiler_params=cp)
    def agg_kernel(table_hbm, src_hbm, dst_hbm, *refs):
        if with_deg:
            out_hbm, deg_hbm = refs[0], refs[1]
            rest = refs[2:-1]
            ldeg = refs[-1]
        else:
            out_hbm = refs[0]
            rest = refs[1:]
        sidx = rest[0:nib]
        didx = rest[nib:2 * nib]
        rows = rest[2 * nib:2 * nib + nbuf]
        shared = rest[2 * nib + nbuf]
        sems = rest[2 * nib + nbuf + 1:]
        semg = sems[0:nbuf]
        semi = sems[nbuf:nbuf + nib]
        semsc = sems[nbuf + nib:]
        cid = lax.axis_index("c")
        sid = lax.axis_index("s")
        wid = sid * 2 + cid
        cbase = wid * niter * chunk

        def idx_copies(j, q):
            base = cbase + j * chunk
            return (pltpu.make_async_copy(src_hbm.at[pl.ds(base, chunk)],
                                          sidx[q], semi[q]),
                    pltpu.make_async_copy(dst_hbm.at[pl.ds(base, chunk)],
                                          didx[q], semi[q]))

        def gather(q, b):
            return pltpu.make_async_copy(table_hbm.at[sidx[q]],
                                         rows[b], semg[b])

        def scatter(q, b):
            return pltpu.make_async_copy(rows[b], shared.at[didx[q]],
                                         semsc[b])

        myreal = jnp.clip(ncr - wid * niter, 0, niter)

        for q in range(nib - 1):
            @pl.when(q < myreal)
            def _():
                for c in idx_copies(q, q):
                    c.start()

        zv = jnp.zeros((_LANES,), jnp.float32)

        @pl.loop(0, chunk)
        def _(i):
            for k in range(width // _LANES):
                rows[0][i, pl.ds(k * _LANES, _LANES)] = zv

        base_r = sid * rows_per_tile
        for f in range(rows_per_tile // chunk):
            pltpu.sync_copy(rows[0],
                            shared.at[pl.ds(base_r + f * chunk, chunk)])
        rem = rows_per_tile % chunk
        if rem:
            pltpu.sync_copy(
                rows[0].at[pl.ds(0, rem)],
                shared.at[pl.ds(base_r + rows_per_tile - rem, rem)])
        if with_deg:
            @pl.loop(0, npad // _LANES)
            def _(i):
                ldeg[pl.ds(i * _LANES, _LANES)] = zv

        plsc.subcore_barrier()

        for js in range(nbuf - 1):
            @pl.when(js < myreal)
            def _():
                for c in idx_copies(js, js):
                    c.wait()
                gather(js, js).start()

        ones = jnp.ones((_LANES,), jnp.float32)
        unroll = nib * nbuf // _gcd(nib, nbuf)

        @pl.loop(0, -(-niter // unroll))
        def _(jj):
            for u in range(unroll):
                j = jj * unroll + u
                q = u % nib
                r = u % nbuf

                @pl.when(j < myreal)
                def _():
                    gather(q, r).wait()
                    scatter(q, r).start(add=True)
                    if with_deg:
                        for k in range(chunk // _LANES):
                            idxv = didx[q][pl.ds(k * _LANES, _LANES)]
                            plsc.addupdate_scatter(ldeg, [idxv], ones)

                    @pl.when(j + nbuf - 1 < myreal)
                    def _():
                        for c in idx_copies(j + nbuf - 1,
                                            (q + nbuf - 1) % nib):
                            c.wait()

                        @pl.when(j >= 1)
                        def _():
                            scatter((q + nib - 1) % nib,
                                    (q + nbuf - 1) % nbuf).wait()

                        gather((q + nbuf - 1) % nib,
                               (q + nbuf - 1) % nbuf).start()

                        @pl.when(j + nib - 1 < myreal)
                        def _():
                            for c in idx_copies(j + nib - 1,
                                                (q + nib - 1) % nib):
                                c.start()

        for s in range(nbuf):
            @pl.when(myreal > s)
            def _():
                scatter(0, s).wait()

        plsc.subcore_barrier()
        pltpu.sync_copy(shared.at[pl.ds(base_r, rows_per_tile)],
                        out_hbm.at[pl.ds(cid * npad + base_r, rows_per_tile)])
        if with_deg:
            pltpu.sync_copy(ldeg, deg_hbm.at[wid])

    return agg_kernel(table, src, dst)


def _tc_layer1_body(pa_ref, dp_ref, w_ref, b_ref, h_ref, dinv_ref):
    npad = pa_ref.shape[0] // 2
    s = pa_ref[:npad, :] + pa_ref[npad:, :]
    ones = jnp.ones((_NTILES, 1), jnp.float32)
    deg = lax.dot_general(dp_ref[...], ones, (((0,), (0,)), ((), ())),
                          preferred_element_type=jnp.float32)
    dinv = 1.0 / jnp.maximum(deg, 1.0)
    dinv_ref[...] = dinv
    z = jnp.dot(s * dinv, w_ref[...], preferred_element_type=jnp.float32)
    h_ref[...] = jnp.maximum(z + b_ref[...], 0.0)


def _tc_layer2_body(pb_ref, dinv_ref, batch_ref, w_ref, b_ref, wo_ref, bo_ref,
                    out_ref, *, num_graphs):
    npad = pb_ref.shape[0] // 2
    s = pb_ref[:npad, :] + pb_ref[npad:, :]
    h = jnp.maximum(
        jnp.dot(s * dinv_ref[...], w_ref[...],
                preferred_element_type=jnp.float32) + b_ref[...], 0.0)
    npd = batch_ref.shape[0]
    b = batch_ref[...].reshape(1, npd)
    gids = lax.broadcasted_iota(jnp.int32, (num_graphs, 1), 0)
    pt = (b == gids).astype(jnp.float32)
    counts = jnp.maximum(jnp.sum(pt, axis=1, keepdims=True), 1.0)
    hg = jnp.dot(pt, h, preferred_element_type=jnp.float32) / counts
    out_ref[...] = jnp.dot(hg, wo_ref[...],
                           preferred_element_type=jnp.float32) + bo_ref[...]


def kernel(x, edge_index, batch, W1, b1, W2, b2, Wout, bout):
    n, d = x.shape
    num_graphs = 64
    npad = ((n + _NTILES * 8 - 1) // (_NTILES * 8)) * (_NTILES * 8)

    src = edge_index[0]
    dst = edge_index[1]

    pa, dp = _sc_edge_aggregate(x, src, dst, npad, n, with_deg=True,
                                chunk=80, nbuf=3, nib=6)
    h1, dinv = pl.pallas_call(
        _tc_layer1_body,
        out_shape=[jax.ShapeDtypeStruct((npad, 128), jnp.float32),
                   jax.ShapeDtypeStruct((npad, 1), jnp.float32)],
    )(pa, dp, W1, b1)

    (pb,) = _sc_edge_aggregate(h1, src, dst, npad, n, with_deg=False,
                               chunk=80, nbuf=3, nib=6)

    batch_p = jnp.concatenate(
        [batch, jnp.full((npad - n,), num_graphs, jnp.int32)])
    out = pl.pallas_call(
        functools.partial(_tc_layer2_body, num_graphs=num_graphs),
        out_shape=jax.ShapeDtypeStruct((num_graphs, 128), jnp.float32),
    )(pb, dinv, batch_p, W2, b2, Wout, bout)
    return out


# speedup vs baseline: 1.0784x; 1.0513x over previous
import dataclasses
import functools
from math import gcd as _gcd

import jax
import jax.numpy as jnp
from jax import lax
from jax.experimental import pallas as pl
from jax.experimental.pallas import tpu as pltpu
from jax.experimental.pallas import tpu_sc as plsc

_CHUNK = 80
_NTILES = 32
_SUBCORES = 16
_LANES = 16


def _sc_edge_aggregate(table, src, dst, npad, n, with_deg, chunk, nbuf, nib):
    v, width = table.shape
    e = src.shape[0]
    ncr = -(-e // chunk)
    niter = -(-ncr // _NTILES)
    if ncr * chunk > e:
        pad = ncr * chunk - e
        src = jnp.concatenate([src, jnp.zeros((pad,), jnp.int32)])
        dst = jnp.concatenate([dst, jnp.full((pad,), n + 8, jnp.int32)])
    rows_per_tile = npad // _SUBCORES

    mesh = plsc.VectorSubcoreMesh(core_axis_name="c", subcore_axis_name="s")

    out_type = [jax.ShapeDtypeStruct((2 * npad, width), jnp.float32)]
    scratch = [pltpu.VMEM((chunk,), jnp.int32)] * (2 * nib)
    scratch += [pltpu.VMEM((chunk, width), jnp.float32)] * nbuf
    scratch += [
        pltpu.VMEM_SHARED((npad, width), jnp.float32),
    ]
    scratch += [pltpu.SemaphoreType.DMA] * (nbuf + nib + nbuf)
    if with_deg:
        out_type.append(jax.ShapeDtypeStruct((_NTILES, npad), jnp.float32))
        scratch.append(pltpu.VMEM((npad,), jnp.float32))

    cp = pltpu.CompilerParams()
    if "needs_layout_passes" in pltpu.CompilerParams.__dataclass_fields__:
        cp = dataclasses.replace(cp, needs_layout_passes=False)

    @functools.partial(pl.kernel, out_type=out_type, mesh=mesh,
                       scratch_types=scratch, compiler_params=cp)
    def agg_kernel(table_hbm, src_hbm, dst_hbm, *refs):
        if with_deg:
            out_hbm, deg_hbm = refs[0], refs[1]
            rest = refs[2:-1]
            ldeg = refs[-1]
        else:
            out_hbm = refs[0]
            rest = refs[1:]
        sidx = rest[0:nib]
        didx = rest[nib:2 * nib]
        rows = rest[2 * nib:2 * nib + nbuf]
        shared = rest[2 * nib + nbuf]
        sems = rest[2 * nib + nbuf + 1:]
        semg = sems[0:nbuf]
        semi = sems[nbuf:nbuf + nib]
        semsc = sems[nbuf + nib:]
        cid = lax.axis_index("c")
        sid = lax.axis_index("s")
        wid = sid * 2 + cid
        cbase = wid * niter * chunk

        def idx_copies(j, q):
            base = cbase + j * chunk
            return (pltpu.make_async_copy(src_hbm.at[pl.ds(base, chunk)],
                                          sidx[q], semi[q]),
                    pltpu.make_async_copy(dst_hbm.at[pl.ds(base, chunk)],
                                          didx[q], semi[q]))

        def gather(q, b):
            return pltpu.make_async_copy(table_hbm.at[sidx[q]],
                                         rows[b], semg[b])

        def scatter(q, b):
            return pltpu.make_async_copy(rows[b], shared.at[didx[q]],
                                         semsc[b])

        myreal = jnp.clip(ncr - wid * niter, 0, niter)

        for q in range(nib - 1):
            @pl.when(q < myreal)
            def _():
                for c in idx_copies(q, q):
                    c.start()

        zv = jnp.zeros((_LANES,), jnp.float32)

        @pl.loop(0, chunk)
        def _(i):
            for k in range(width // _LANES):
                rows[0][i, pl.ds(k * _LANES, _LANES)] = zv

        base_r = sid * rows_per_tile
        for f in range(rows_per_tile // chunk):
            pltpu.sync_copy(rows[0],
                            shared.at[pl.ds(base_r + f * chunk, chunk)])
        rem = rows_per_tile % chunk
        if rem:
            pltpu.sync_copy(
                rows[0].at[pl.ds(0, rem)],
                shared.at[pl.ds(base_r + rows_per_tile - rem, rem)])
        if with_deg:
            @pl.loop(0, npad // _LANES)
            def _(i):
                ldeg[pl.ds(i * _LANES, _LANES)] = zv

        plsc.subcore_barrier()

        for js in range(nbuf - 1):
            @pl.when(js < myreal)
            def _():
                for c in idx_copies(js, js):
                    c.wait()
                gather(js, js).start()

        ones = jnp.ones((_LANES,), jnp.float32)
        unroll = nib * nbuf // _gcd(nib, nbuf)

        @pl.loop(0, -(-niter // unroll))
        def _(jj):
            for u in range(unroll):
                j = jj * unroll + u
                q = u % nib
                r = u % nbuf

                @pl.when(j < myreal)
                def _():
                    gather(q, r).wait()
                    scatter(q, r).start(add=True)
                    if with_deg:
                        for k in range(chunk // _LANES):
                            idxv = didx[q][pl.ds(k * _LANES, _LANES)]
                            plsc.addupdate_scatter(ldeg, [idxv], ones)

                    @pl.when(j + nbuf - 1 < myreal)
                    def _():
                        for c in idx_copies(j + nbuf - 1,
                                            (q + nbuf - 1) % nib):
                            c.wait()

                        @pl.when(j >= 1)
                        def _():
                            scatter((q + nib - 1) % nib,
                                    (q + nbuf - 1) % nbuf).wait()

                        gather((q + nbuf - 1) % nib,
                               (q + nbuf - 1) % nbuf).start()

                        @pl.when(j + nib - 1 < myreal)
                        def _():
                            for c in idx_copies(j + nib - 1,
                                                (q + nib - 1) % nib):
                                c.start()

        for s in range(nbuf):
            @pl.when(myreal > s)
            def _():
                scatter(0, s).wait()

        plsc.subcore_barrier()
        pltpu.sync_copy(shared.at[pl.ds(base_r, rows_per_tile)],
                        out_hbm.at[pl.ds(cid * npad + base_r, rows_per_tile)])
        if with_deg:
            pltpu.sync_copy(ldeg, deg_hbm.at[wid])

    return agg_kernel(table, src, dst)


def _tc_split_body(ei_ref, src_ref, dst_ref):
    src_ref[...] = ei_ref[0, :]
    dst_ref[...] = ei_ref[1, :]


def _tc_layer1_body(pa_ref, dp_ref, w_ref, b_ref, h_ref, dinv_ref):
    npad = pa_ref.shape[0] // 2
    s = pa_ref[:npad, :] + pa_ref[npad:, :]
    ones = jnp.ones((_NTILES, 1), jnp.float32)
    deg = lax.dot_general(dp_ref[...], ones, (((0,), (0,)), ((), ())),
                          preferred_element_type=jnp.float32)
    dinv = 1.0 / jnp.maximum(deg, 1.0)
    dinv_ref[...] = dinv
    z = jnp.dot(s * dinv, w_ref[...], preferred_element_type=jnp.float32)
    h_ref[...] = jnp.maximum(z + b_ref[...], 0.0)


def _tc_layer2_body(pb_ref, dinv_ref, batch_ref, w_ref, b_ref, wo_ref, bo_ref,
                    out_ref, *, num_graphs):
    npad = pb_ref.shape[0] // 2
    s = pb_ref[:npad, :] + pb_ref[npad:, :]
    h = jnp.maximum(
        jnp.dot(s * dinv_ref[...], w_ref[...],
                preferred_element_type=jnp.float32) + b_ref[...], 0.0)
    npd = batch_ref.shape[0]
    b = batch_ref[...].reshape(1, npd)
    gids = lax.broadcasted_iota(jnp.int32, (num_graphs, 1), 0)
    pt = (b == gids).astype(jnp.float32)
    counts = jnp.maximum(jnp.sum(pt, axis=1, keepdims=True), 1.0)
    hg = jnp.dot(pt, h, preferred_element_type=jnp.float32) / counts
    out_ref[...] = jnp.dot(hg, wo_ref[...],
                           preferred_element_type=jnp.float32) + bo_ref[...]


def kernel(x, edge_index, batch, W1, b1, W2, b2, Wout, bout):
    n, d = x.shape
    num_graphs = 64
    npad = ((n + _NTILES * 8 - 1) // (_NTILES * 8)) * (_NTILES * 8)

    e = edge_index.shape[1]
    src, dst = pl.pallas_call(
        _tc_split_body,
        out_shape=[jax.ShapeDtypeStruct((e,), jnp.int32),
                   jax.ShapeDtypeStruct((e,), jnp.int32)],
    )(edge_index)

    pa, dp = _sc_edge_aggregate(x, src, dst, npad, n, with_deg=True,
                                chunk=80, nbuf=3, nib=6)
    h1, dinv = pl.pallas_call(
        _tc_layer1_body,
        out_shape=[jax.ShapeDtypeStruct((npad, 128), jnp.float32),
                   jax.ShapeDtypeStruct((npad, 1), jnp.float32)],
    )(pa, dp, W1, b1)

    (pb,) = _sc_edge_aggregate(h1, src, dst, npad, n, with_deg=False,
                               chunk=80, nbuf=3, nib=6)

    batch_p = jnp.concatenate(
        [batch, jnp.full((npad - n,), num_graphs, jnp.int32)])
    out = pl.pallas_call(
        functools.partial(_tc_layer2_body, num_graphs=num_graphs),
        out_shape=jax.ShapeDtypeStruct((num_graphs, 128), jnp.float32),
    )(pb, dinv, batch_p, W2, b2, Wout, bout)
    return out
